# baseline (speedup 1.0000x reference)
import functools

import jax
import jax.numpy as jnp
from jax import lax
from jax.experimental import pallas as pl
from jax.experimental.pallas import tpu as pltpu
from jax.experimental.pallas import tpu_sc as plsc

N_NODES = 10000
N_EDGES = 320000
D_FEAT = 128
D_HIDDEN = 256

NC = 2
NS = 16
NW = NC * NS
CHUNK = 128
N_CHUNKS = N_EDGES // CHUNK
N_CHUNKS_PAD = 2560
MY_CHUNKS = N_CHUNKS_PAD // NW
N_PAD = 10240
ROWS_PER_TILE = N_PAD // NS
ZROWS = 128


def _sc_body(feat_hbm, edges_hbm, acc_out, deg_out,
             idxbuf, rows, deg_local, acc_shared, isem, gsem, ssem):
  c = lax.axis_index("c")
  s = lax.axis_index("s")
  wid = s * NC + c

  zeros16 = jnp.zeros((16,), jnp.float32)

  def zero_deg(j, _):
    deg_local[pl.ds(j * 16, 16)] = zeros16
    return 0

  lax.fori_loop(0, N_PAD // 16, zero_deg, 0)

  def zero_rows(j, _):
    for k in range(D_FEAT // 16):
      rows[0, j, pl.ds(k * 16, 16)] = zeros16
    return 0

  lax.fori_loop(0, ZROWS, zero_rows, 0)

  row0 = s * ROWS_PER_TILE
  for k in range(ROWS_PER_TILE // ZROWS):
    pltpu.sync_copy(rows.at[0], acc_shared.at[pl.ds(row0 + k * ZROWS, ZROWS)])
  plsc.subcore_barrier()

  ones16 = jnp.ones((16,), jnp.float32)

  def chunk_of(i):
    return wid + i * NW

  def idx_issue(i, slot):
    pltpu.async_copy(edges_hbm.at[chunk_of(i)], idxbuf.at[slot], isem.at[slot])

  def idx_wait(i, slot):
    pltpu.make_async_copy(edges_hbm.at[chunk_of(i)], idxbuf.at[slot],
                          isem.at[slot]).wait()

  def gather_issue(i, x, slot):
    pltpu.async_copy(feat_hbm.at[idxbuf.at[slot, 0]], rows.at[x], gsem.at[x])

  def gather_wait(i, x, slot):
    pltpu.make_async_copy(feat_hbm.at[idxbuf.at[slot, 0]], rows.at[x],
                          gsem.at[x]).wait()

  def scatter_issue(i, x, slot):
    pltpu.async_copy(rows.at[x], acc_shared.at[idxbuf.at[slot, 1]],
                     ssem.at[x], add=True)

  def scatter_wait(i, x, slot):
    pltpu.make_async_copy(rows.at[x], acc_shared.at[idxbuf.at[slot, 1]],
                          ssem.at[x]).wait()

  def deg_update(slot):
    for j in range(CHUNK // 16):
      d16 = idxbuf[slot, 1, pl.ds(j * 16, 16)]
      plsc.addupdate_scatter(deg_local, [d16], ones16)

  def sub_step(i, r):
    x, xo = r % 2, (r + 1) % 2
    if isinstance(i, int):
      first, last = i < 2, i >= MY_CHUNKS - 2
    else:
      first, last = False, False
    if not first:
      scatter_wait(i - 2, x, r % 4)
    if not last:
      idx_issue(i + 2, (r + 2) % 4)
    idx_wait(i, r % 4)
    gather_issue(i, x, r % 4)
    if not (isinstance(i, int) and i == 0):
      deg_update((r + 3) % 4)
      gather_wait(i - 1, xo, (r + 3) % 4)
      scatter_issue(i - 1, xo, (r + 3) % 4)

  idx_issue(0, 0)
  idx_issue(1, 1)
  for i in range(4):
    sub_step(i, i)

  def q_body(q, _):
    i0 = 4 * q
    for r in range(4):
      sub_step(i0 + r, r)
    return 0

  lax.fori_loop(1, MY_CHUNKS // 4 - 1, q_body, 0)

  for i in range(MY_CHUNKS - 4, MY_CHUNKS):
    sub_step(i, i % 4)

  deg_update((MY_CHUNKS - 1) % 4)
  gather_wait(MY_CHUNKS - 1, (MY_CHUNKS - 1) % 2, (MY_CHUNKS - 1) % 4)
  scatter_issue(MY_CHUNKS - 1, (MY_CHUNKS - 1) % 2, (MY_CHUNKS - 1) % 4)
  scatter_wait(MY_CHUNKS - 2, (MY_CHUNKS - 2) % 2, (MY_CHUNKS - 2) % 4)
  scatter_wait(MY_CHUNKS - 1, (MY_CHUNKS - 1) % 2, (MY_CHUNKS - 1) % 4)
  plsc.subcore_barrier()

  pltpu.sync_copy(deg_local, deg_out.at[wid, 0])
  for k in range(ROWS_PER_TILE // ZROWS):
    r = row0 + k * ZROWS
    pltpu.sync_copy(acc_shared.at[pl.ds(r, ZROWS)], acc_out.at[c, pl.ds(r, ZROWS)])


@jax.jit
def _sc_aggregate(features, edges):
  mesh = plsc.VectorSubcoreMesh(core_axis_name="c", subcore_axis_name="s")
  return pl.kernel(
      _sc_body,
      out_type=[
          jax.ShapeDtypeStruct((NC, N_PAD, D_FEAT), jnp.float32),
          jax.ShapeDtypeStruct((NW, 1, N_PAD), jnp.float32),
      ],
      mesh=mesh,
      compiler_params=pltpu.CompilerParams(needs_layout_passes=False),
      scratch_types=[
          pltpu.VMEM((4, 2, CHUNK), jnp.int32),
          pltpu.VMEM((2, CHUNK, D_FEAT), jnp.float32),
          pltpu.VMEM((N_PAD,), jnp.float32),
          pltpu.VMEM_SHARED((N_PAD, D_FEAT), jnp.float32),
          pltpu.SemaphoreType.DMA((4,)),
          pltpu.SemaphoreType.DMA((2,)),
          pltpu.SemaphoreType.DMA((2,)),
      ],
  )(features, edges)


def _tc_head_body(acc_ref, deg_ref, wmp_ref, bmp_ref, w1_ref, b1_ref,
                  w2_ref, b2_ref, out_ref):
  acc = acc_ref[0] + acc_ref[1]
  deg = jnp.sum(deg_ref[0], axis=0)
  deg = jnp.maximum(deg, 1.0)
  h = acc / deg[:, None]
  h = jnp.maximum(jnp.dot(h, wmp_ref[...], preferred_element_type=jnp.float32)
                  + bmp_ref[...], 0.0)
  h = jnp.maximum(jnp.dot(h, w1_ref[...], preferred_element_type=jnp.float32)
                  + b1_ref[...], 0.0)
  out_ref[...] = (jnp.dot(h, w2_ref[...], preferred_element_type=jnp.float32)
                  + b2_ref[...])


@jax.jit
def _tc_head(acc2, deg32, W_mp, b_mp, W1, b1, W2, b2):
  R = 1000
  grid = (N_NODES // R,)
  f = pl.pallas_call(
      _tc_head_body,
      grid=grid,
      in_specs=[
          pl.BlockSpec((NC, R, D_FEAT), lambda i: (0, i, 0)),
          pl.BlockSpec((1, NW, R), lambda i: (i, 0, 0)),
          pl.BlockSpec((D_FEAT, D_FEAT), lambda i: (0, 0)),
          pl.BlockSpec((1, D_FEAT), lambda i: (0, 0)),
          pl.BlockSpec((D_FEAT, D_HIDDEN), lambda i: (0, 0)),
          pl.BlockSpec((1, D_HIDDEN), lambda i: (0, 0)),
          pl.BlockSpec((D_HIDDEN, D_FEAT), lambda i: (0, 0)),
          pl.BlockSpec((1, D_FEAT), lambda i: (0, 0)),
      ],
      out_specs=pl.BlockSpec((R, D_FEAT), lambda i: (i, 0)),
      out_shape=jax.ShapeDtypeStruct((N_NODES, D_FEAT), jnp.float32),
  )
  deg_t = deg32.reshape(NW, N_NODES // R, R).transpose(1, 0, 2)
  return f(acc2, deg_t, W_mp, b_mp, W1, b1, W2, b2)


def kernel(features, edge_index, W_mp, b_mp, W1, b1, W2, b2):
  src = edge_index[0].astype(jnp.int32)
  dst = edge_index[1].astype(jnp.int32)
  n_extra = N_CHUNKS_PAD * CHUNK - N_EDGES
  srcp = jnp.concatenate([src, jnp.zeros((n_extra,), jnp.int32)])
  dstp = jnp.concatenate([dst, jnp.full((n_extra,), N_PAD - 1, jnp.int32)])
  edges = jnp.stack([srcp.reshape(N_CHUNKS_PAD, CHUNK),
                     dstp.reshape(N_CHUNKS_PAD, CHUNK)], axis=1)
  acc2, deg32 = _sc_aggregate(features, edges)
  deg32 = deg32.reshape(NW, N_PAD)[:, :N_NODES]
  return _tc_head(acc2, deg32, W_mp, b_mp.reshape(1, -1),
                  W1, b1.reshape(1, -1), W2, b2.reshape(1, -1))
